# baseline (speedup 1.0000x reference)
import functools

import jax
import jax.numpy as jnp
from jax import lax
from jax.experimental import pallas as pl
from jax.experimental.pallas import tpu as pltpu
from jax.experimental.pallas import tpu_sc as plsc

NC = 2
NS = 16
LANES = 16
NW = NC * NS
CHUNK = 128
NBUF = 13


def _sc_mesh():
    return plsc.VectorSubcoreMesh(
        core_axis_name="c", subcore_axis_name="s",
        num_cores=NC, num_subcores=NS)


_SC_PARAMS = pltpu.CompilerParams(use_tc_tiling_on_sc=False)


def _make_deg_kernel(n_pad, epw):
    rpt = n_pad // NS
    nfull = epw // CHUNK
    tail = epw - nfull * CHUNK

    @functools.partial(
        pl.kernel,
        out_type=jax.ShapeDtypeStruct((NC * n_pad,), jnp.float32),
        mesh=_sc_mesh(),
        compiler_params=_SC_PARAMS,
        scratch_types=[
            pltpu.VMEM((epw,), jnp.int32),
            pltpu.VMEM((CHUNK,), jnp.float32),
            pltpu.VMEM((rpt,), jnp.float32),
            pltpu.VMEM_SHARED((n_pad,), jnp.float32),
            pltpu.SemaphoreType.DMA,
        ],
    )
    def deg_kernel(col_hbm, out_hbm, colv, ones_v, zero_v, acc_sh, sem):
        cid = lax.axis_index("c")
        sid = lax.axis_index("s")
        wid = sid * NC + cid

        def fill_zero(i, carry):
            zero_v[pl.ds(i * LANES, LANES)] = jnp.zeros((LANES,), jnp.float32)
            return carry
        lax.fori_loop(0, rpt // LANES, fill_zero, 0)

        def fill_ones(i, carry):
            ones_v[pl.ds(i * LANES, LANES)] = jnp.ones((LANES,), jnp.float32)
            return carry
        lax.fori_loop(0, CHUNK // LANES, fill_ones, 0)

        pltpu.sync_copy(zero_v, acc_sh.at[pl.ds(sid * rpt, rpt)])
        plsc.subcore_barrier()

        pltpu.sync_copy(col_hbm.at[pl.ds(wid * epw, epw)], colv)

        if tail:
            pltpu.sync_copy(ones_v.at[pl.ds(0, tail)],
                            acc_sh.at[colv.at[pl.ds(nfull * CHUNK, tail)]],
                            add=True)

        def body(j, carry):
            pltpu.async_copy(ones_v,
                             acc_sh.at[colv.at[pl.ds(j * CHUNK, CHUNK)]],
                             sem, add=True)
            return carry
        lax.fori_loop(0, nfull, body, 0)

        def drain(j, carry):
            pltpu.make_async_copy(
                ones_v, acc_sh.at[colv.at[pl.ds(j * CHUNK, CHUNK)]],
                sem).wait()
            return carry
        lax.fori_loop(0, nfull, drain, 0)

        plsc.subcore_barrier()
        pltpu.sync_copy(acc_sh.at[pl.ds(sid * rpt, rpt)],
                        out_hbm.at[pl.ds(cid * n_pad + sid * rpt, rpt)])

    return deg_kernel


def _make_agg_kernel(n_pad, epw):
    rpt = n_pad // NS
    nfull = epw // CHUNK
    tail = epw - nfull * CHUNK
    assert nfull % NBUF == 0 and nfull // NBUF >= 2

    @functools.partial(
        pl.kernel,
        out_type=jax.ShapeDtypeStruct((NC * n_pad, LANES), jnp.float32),
        mesh=_sc_mesh(),
        compiler_params=_SC_PARAMS,
        scratch_types=(
            [pltpu.VMEM((epw,), jnp.int32)] * 2
            + [pltpu.VMEM((CHUNK, LANES), jnp.float32)] * NBUF
            + [pltpu.VMEM((max(tail, 1), LANES), jnp.float32)]
            + [pltpu.VMEM((rpt, LANES), jnp.float32)]
            + [pltpu.VMEM_SHARED((n_pad, LANES), jnp.float32)]
            + [pltpu.SemaphoreType.DMA] * (2 * NBUF + 1)
        ),
    )
    def agg_kernel(edge_hbm, z_hbm, out_hbm, *refs):
        rowv, colv = refs[0], refs[1]
        bufs = refs[2:2 + NBUF]
        tbuf = refs[2 + NBUF]
        zero_v = refs[3 + NBUF]
        acc_sh = refs[4 + NBUF]
        gsem = refs[5 + NBUF:5 + 2 * NBUF]
        ssem = refs[5 + 2 * NBUF:5 + 3 * NBUF]
        tsem = refs[5 + 3 * NBUF]

        cid = lax.axis_index("c")
        sid = lax.axis_index("s")
        wid = sid * NC + cid

        def fill_zero(i, carry):
            zero_v[i, :] = jnp.zeros((LANES,), jnp.float32)
            return carry
        lax.fori_loop(0, rpt, fill_zero, 0)

        pltpu.sync_copy(zero_v, acc_sh.at[pl.ds(sid * rpt, rpt)])
        plsc.subcore_barrier()

        pltpu.sync_copy(edge_hbm.at[0, pl.ds(wid * epw, epw)], rowv)
        pltpu.sync_copy(edge_hbm.at[1, pl.ds(wid * epw, epw)], colv)

        def start_g(j, b):
            pltpu.async_copy(z_hbm.at[rowv.at[pl.ds(j * CHUNK, CHUNK)]],
                             bufs[b], gsem[b])

        def wait_g(j, b):
            pltpu.make_async_copy(z_hbm.at[rowv.at[pl.ds(j * CHUNK, CHUNK)]],
                                  bufs[b], gsem[b]).wait()

        def start_s(j, b):
            pltpu.async_copy(bufs[b],
                             acc_sh.at[colv.at[pl.ds(j * CHUNK, CHUNK)]],
                             ssem[b], add=True)

        def wait_s(j, b):
            pltpu.make_async_copy(bufs[b],
                                  acc_sh.at[colv.at[pl.ds(j * CHUNK, CHUNK)]],
                                  ssem[b]).wait()

        if tail:
            toff = nfull * CHUNK
            pltpu.async_copy(z_hbm.at[rowv.at[pl.ds(toff, tail)]],
                             tbuf, tsem).wait()
            pltpu.sync_copy(tbuf, acc_sh.at[colv.at[pl.ds(toff, tail)]],
                            add=True)

        for b in range(NBUF):
            start_g(b, b)

        def body(ii, carry):
            j0 = ii * NBUF
            for b in range(NBUF):
                wait_g(j0 + b, b)
                start_s(j0 + b, b)
            for b in range(NBUF):
                wait_s(j0 + b, b)
                start_g(j0 + NBUF + b, b)
            return carry
        lax.fori_loop(0, nfull // NBUF - 1, body, 0)

        j0 = nfull - NBUF
        for b in range(NBUF):
            wait_g(j0 + b, b)
            start_s(j0 + b, b)
        for b in range(NBUF):
            wait_s(j0 + b, b)

        plsc.subcore_barrier()
        pltpu.sync_copy(acc_sh.at[pl.ds(sid * rpt, rpt)],
                        out_hbm.at[pl.ds(cid * n_pad + sid * rpt, rpt)])

    return agg_kernel


def kernel(x, edge_index, W, b):
    n, f_in = x.shape
    f_out = W.shape[0]
    e = edge_index.shape[1]

    n_pad = ((n + 1 + NS * LANES - 1) // (NS * LANES)) * (NS * LANES)
    assert e % (NW * 8) == 0
    epw = e // NW

    def tc0_body(x_ref, w_ref, y_ref):
        y_ref[...] = lax.dot_general(x_ref[...], w_ref[...],
                                     (((1,), (1,)), ((), ())),
                                     preferred_element_type=jnp.float32)

    y = pl.pallas_call(
        tc0_body,
        out_shape=jax.ShapeDtypeStruct((n, f_out), jnp.float32),
    )(x, W)

    degp = _make_deg_kernel(n_pad, epw)(edge_index[1])

    def tc1_body(y_ref, degp_ref, z_ref):
        cnt = sum(degp_ref[pl.ds(c * n_pad, n)] for c in range(NC))
        dis = lax.rsqrt(cnt + 1.0)
        z_ref[...] = y_ref[...] * lax.broadcast_in_dim(dis, (n, f_out), (0,))

    z = pl.pallas_call(
        tc1_body,
        out_shape=jax.ShapeDtypeStruct((n, f_out), jnp.float32),
    )(y, degp)

    aggp = _make_agg_kernel(n_pad, epw)(edge_index, z)

    def tc2_body(aggp_ref, z_ref, degp_ref, b_ref, o_ref):
        agg = sum(aggp_ref[c * n_pad:c * n_pad + n, :] for c in range(NC))
        cnt = sum(degp_ref[pl.ds(c * n_pad, n)] for c in range(NC))
        dis = lax.broadcast_in_dim(lax.rsqrt(cnt + 1.0), (n, f_out), (0,))
        h = dis * (agg + z_ref[...]) + b_ref[...]
        m = jnp.max(h, axis=1, keepdims=True)
        sh = h - m
        lse = jnp.log(jnp.sum(jnp.exp(sh), axis=1, keepdims=True))
        o_ref[...] = sh - lse

    return pl.pallas_call(
        tc2_body,
        out_shape=jax.ShapeDtypeStruct((n, f_out), jnp.float32),
    )(aggp, z, degp, b.reshape(1, f_out))


# speedup vs baseline: 1.1639x; 1.1639x over previous
import functools

import jax
import jax.numpy as jnp
from jax import lax
from jax.experimental import pallas as pl
from jax.experimental.pallas import tpu as pltpu
from jax.experimental.pallas import tpu_sc as plsc

NC = 2
NS = 16
LANES = 16
NW = NC * NS
CHUNK = 128
NBUF = 13


def _sc_mesh():
    return plsc.VectorSubcoreMesh(
        core_axis_name="c", subcore_axis_name="s",
        num_cores=NC, num_subcores=NS)


_SC_PARAMS = pltpu.CompilerParams(use_tc_tiling_on_sc=False)


def _make_deg_kernel(n_pad, epw):
    rpt = n_pad // NS
    nfull = epw // CHUNK
    tail = epw - nfull * CHUNK

    @functools.partial(
        pl.kernel,
        out_type=jax.ShapeDtypeStruct((NC * n_pad,), jnp.float32),
        mesh=_sc_mesh(),
        compiler_params=_SC_PARAMS,
        scratch_types=[
            pltpu.VMEM((epw,), jnp.int32),
            pltpu.VMEM((CHUNK,), jnp.float32),
            pltpu.VMEM((rpt,), jnp.float32),
            pltpu.VMEM_SHARED((n_pad,), jnp.float32),
            pltpu.SemaphoreType.DMA,
        ],
    )
    def deg_kernel(edge_hbm, out_hbm, colv, ones_v, zero_v, acc_sh, sem):
        cid = lax.axis_index("c")
        sid = lax.axis_index("s")
        wid = sid * NC + cid

        def fill_zero(i, carry):
            zero_v[pl.ds(i * LANES, LANES)] = jnp.zeros((LANES,), jnp.float32)
            return carry
        lax.fori_loop(0, rpt // LANES, fill_zero, 0)

        def fill_ones(i, carry):
            ones_v[pl.ds(i * LANES, LANES)] = jnp.ones((LANES,), jnp.float32)
            return carry
        lax.fori_loop(0, CHUNK // LANES, fill_ones, 0)

        pltpu.sync_copy(zero_v, acc_sh.at[pl.ds(sid * rpt, rpt)])
        plsc.subcore_barrier()

        pltpu.sync_copy(edge_hbm.at[1, pl.ds(wid * epw, epw)], colv)

        if tail:
            pltpu.sync_copy(ones_v.at[pl.ds(0, tail)],
                            acc_sh.at[colv.at[pl.ds(nfull * CHUNK, tail)]],
                            add=True)

        def body(j, carry):
            pltpu.async_copy(ones_v,
                             acc_sh.at[colv.at[pl.ds(j * CHUNK, CHUNK)]],
                             sem, add=True)
            return carry
        lax.fori_loop(0, nfull, body, 0)

        def drain(j, carry):
            pltpu.make_async_copy(
                ones_v, acc_sh.at[colv.at[pl.ds(j * CHUNK, CHUNK)]],
                sem).wait()
            return carry
        lax.fori_loop(0, nfull, drain, 0)

        plsc.subcore_barrier()
        pltpu.sync_copy(acc_sh.at[pl.ds(sid * rpt, rpt)],
                        out_hbm.at[pl.ds(cid * n_pad + sid * rpt, rpt)])

    return deg_kernel


def _make_agg_kernel(n_pad, epw):
    rpt = n_pad // NS
    nfull = epw // CHUNK
    tail = epw - nfull * CHUNK
    assert nfull % NBUF == 0 and nfull // NBUF >= 2

    @functools.partial(
        pl.kernel,
        out_type=jax.ShapeDtypeStruct((NC * n_pad, LANES), jnp.float32),
        mesh=_sc_mesh(),
        compiler_params=_SC_PARAMS,
        scratch_types=(
            [pltpu.VMEM((epw,), jnp.int32)] * 2
            + [pltpu.VMEM((CHUNK, LANES), jnp.float32)] * NBUF
            + [pltpu.VMEM((max(tail, 1), LANES), jnp.float32)]
            + [pltpu.VMEM((rpt, LANES), jnp.float32)]
            + [pltpu.VMEM_SHARED((n_pad, LANES), jnp.float32)]
            + [pltpu.SemaphoreType.DMA] * (2 * NBUF + 1)
        ),
    )
    def agg_kernel(edge_hbm, z_hbm, out_hbm, *refs):
        rowv, colv = refs[0], refs[1]
        bufs = refs[2:2 + NBUF]
        tbuf = refs[2 + NBUF]
        zero_v = refs[3 + NBUF]
        acc_sh = refs[4 + NBUF]
        gsem = refs[5 + NBUF:5 + 2 * NBUF]
        ssem = refs[5 + 2 * NBUF:5 + 3 * NBUF]
        tsem = refs[5 + 3 * NBUF]

        cid = lax.axis_index("c")
        sid = lax.axis_index("s")
        wid = sid * NC + cid

        def fill_zero(i, carry):
            zero_v[i, :] = jnp.zeros((LANES,), jnp.float32)
            return carry
        lax.fori_loop(0, rpt, fill_zero, 0)

        pltpu.sync_copy(zero_v, acc_sh.at[pl.ds(sid * rpt, rpt)])
        plsc.subcore_barrier()

        pltpu.sync_copy(edge_hbm.at[0, pl.ds(wid * epw, epw)], rowv)
        pltpu.sync_copy(edge_hbm.at[1, pl.ds(wid * epw, epw)], colv)

        def start_g(j, b):
            pltpu.async_copy(z_hbm.at[rowv.at[pl.ds(j * CHUNK, CHUNK)]],
                             bufs[b], gsem[b])

        def wait_g(j, b):
            pltpu.make_async_copy(z_hbm.at[rowv.at[pl.ds(j * CHUNK, CHUNK)]],
                                  bufs[b], gsem[b]).wait()

        def start_s(j, b):
            pltpu.async_copy(bufs[b],
                             acc_sh.at[colv.at[pl.ds(j * CHUNK, CHUNK)]],
                             ssem[b], add=True)

        def wait_s(j, b):
            pltpu.make_async_copy(bufs[b],
                                  acc_sh.at[colv.at[pl.ds(j * CHUNK, CHUNK)]],
                                  ssem[b]).wait()

        if tail:
            toff = nfull * CHUNK
            pltpu.async_copy(z_hbm.at[rowv.at[pl.ds(toff, tail)]],
                             tbuf, tsem).wait()
            pltpu.sync_copy(tbuf, acc_sh.at[colv.at[pl.ds(toff, tail)]],
                            add=True)

        for b in range(NBUF):
            start_g(b, b)

        def body(ii, carry):
            j0 = ii * NBUF
            for b in range(NBUF):
                wait_g(j0 + b, b)
                start_s(j0 + b, b)
            for b in range(NBUF):
                wait_s(j0 + b, b)
                start_g(j0 + NBUF + b, b)
            return carry
        lax.fori_loop(0, nfull // NBUF - 1, body, 0)

        j0 = nfull - NBUF
        for b in range(NBUF):
            wait_g(j0 + b, b)
            start_s(j0 + b, b)
        for b in range(NBUF):
            wait_s(j0 + b, b)

        plsc.subcore_barrier()
        pltpu.sync_copy(acc_sh.at[pl.ds(sid * rpt, rpt)],
                        out_hbm.at[pl.ds(cid * n_pad + sid * rpt, rpt)])

    return agg_kernel


def kernel(x, edge_index, W, b):
    n, f_in = x.shape
    f_out = W.shape[0]
    e = edge_index.shape[1]

    n_pad = ((n + 1 + NS * LANES - 1) // (NS * LANES)) * (NS * LANES)
    assert e % (NW * 8) == 0
    epw = e // NW

    def tc0_body(x_ref, w_ref, y_ref):
        y_ref[...] = lax.dot_general(x_ref[...], w_ref[...],
                                     (((1,), (1,)), ((), ())),
                                     preferred_element_type=jnp.float32)

    y = pl.pallas_call(
        tc0_body,
        out_shape=jax.ShapeDtypeStruct((n, f_out), jnp.float32),
    )(x, W)

    degp = _make_deg_kernel(n_pad, epw)(edge_index)

    def tc1_body(y_ref, degp_ref, z_ref):
        cnt = sum(degp_ref[pl.ds(c * n_pad, n)] for c in range(NC))
        dis = lax.rsqrt(cnt + 1.0)
        z_ref[...] = y_ref[...] * lax.broadcast_in_dim(dis, (n, f_out), (0,))

    z = pl.pallas_call(
        tc1_body,
        out_shape=jax.ShapeDtypeStruct((n, f_out), jnp.float32),
    )(y, degp)

    aggp = _make_agg_kernel(n_pad, epw)(edge_index, z)

    def tc2_body(aggp_ref, z_ref, degp_ref, b_ref, o_ref):
        agg = sum(aggp_ref[c * n_pad:c * n_pad + n, :] for c in range(NC))
        cnt = sum(degp_ref[pl.ds(c * n_pad, n)] for c in range(NC))
        dis = lax.broadcast_in_dim(lax.rsqrt(cnt + 1.0), (n, f_out), (0,))
        h = dis * (agg + z_ref[...]) + b_ref[...]
        m = jnp.max(h, axis=1, keepdims=True)
        sh = h - m
        lse = jnp.log(jnp.sum(jnp.exp(sh), axis=1, keepdims=True))
        o_ref[...] = sh - lse

    return pl.pallas_call(
        tc2_body,
        out_shape=jax.ShapeDtypeStruct((n, f_out), jnp.float32),
    )(aggp, z, degp, b.reshape(1, f_out))


# speedup vs baseline: 1.1723x; 1.0072x over previous
import functools

import jax
import jax.numpy as jnp
from jax import lax
from jax.experimental import pallas as pl
from jax.experimental.pallas import tpu as pltpu
from jax.experimental.pallas import tpu_sc as plsc

NC = 2
NS = 16
LANES = 16
NW = NC * NS
CHUNK = 128
NBUF = 13


def _sc_mesh():
    return plsc.VectorSubcoreMesh(
        core_axis_name="c", subcore_axis_name="s",
        num_cores=NC, num_subcores=NS)


_SC_PARAMS = pltpu.CompilerParams(use_tc_tiling_on_sc=False)


def _make_deg_kernel(n_pad, epw):
    rpt = n_pad // NS
    nfull = epw // CHUNK
    tail = epw - nfull * CHUNK

    @functools.partial(
        pl.kernel,
        out_type=jax.ShapeDtypeStruct((NC * n_pad,), jnp.float32),
        mesh=_sc_mesh(),
        compiler_params=_SC_PARAMS,
        scratch_types=[
            pltpu.VMEM((epw,), jnp.int32),
            pltpu.VMEM((CHUNK,), jnp.float32),
            pltpu.VMEM((rpt,), jnp.float32),
            pltpu.VMEM_SHARED((n_pad,), jnp.float32),
            pltpu.SemaphoreType.DMA,
        ],
    )
    def deg_kernel(edge_hbm, out_hbm, colv, ones_v, zero_v, acc_sh, sem):
        cid = lax.axis_index("c")
        sid = lax.axis_index("s")
        wid = sid * NC + cid

        def fill_zero(i, carry):
            zero_v[pl.ds(i * LANES, LANES)] = jnp.zeros((LANES,), jnp.float32)
            return carry
        lax.fori_loop(0, rpt // LANES, fill_zero, 0)

        def fill_ones(i, carry):
            ones_v[pl.ds(i * LANES, LANES)] = jnp.ones((LANES,), jnp.float32)
            return carry
        lax.fori_loop(0, CHUNK // LANES, fill_ones, 0)

        pltpu.sync_copy(zero_v, acc_sh.at[pl.ds(sid * rpt, rpt)])
        plsc.subcore_barrier()

        pltpu.sync_copy(edge_hbm.at[1, pl.ds(wid * epw, epw)], colv)

        if tail:
            pltpu.sync_copy(ones_v.at[pl.ds(0, tail)],
                            acc_sh.at[colv.at[pl.ds(nfull * CHUNK, tail)]],
                            add=True)

        def body(j, carry):
            pltpu.async_copy(ones_v,
                             acc_sh.at[colv.at[pl.ds(j * CHUNK, CHUNK)]],
                             sem, add=True)
            return carry
        lax.fori_loop(0, nfull, body, 0)

        def drain(j, carry):
            pltpu.make_async_copy(
                ones_v, acc_sh.at[colv.at[pl.ds(j * CHUNK, CHUNK)]],
                sem).wait()
            return carry
        lax.fori_loop(0, nfull, drain, 0)

        plsc.subcore_barrier()
        pltpu.sync_copy(acc_sh.at[pl.ds(sid * rpt, rpt)],
                        out_hbm.at[pl.ds(cid * n_pad + sid * rpt, rpt)])

    return deg_kernel


def _make_agg_kernel(n_pad, epw, nz):
    rpt = n_pad // NS
    nfull = epw // CHUNK
    tail = epw - nfull * CHUNK
    assert nfull % NBUF == 0 and nfull // NBUF >= 2

    @functools.partial(
        pl.kernel,
        out_type=jax.ShapeDtypeStruct((NC * n_pad, LANES), jnp.float32),
        mesh=_sc_mesh(),
        compiler_params=_SC_PARAMS,
        scratch_types=(
            [pltpu.VMEM((epw,), jnp.int32)] * 2
            + [pltpu.VMEM((CHUNK, LANES), jnp.float32)] * NBUF
            + [pltpu.VMEM((max(tail, 1), LANES), jnp.float32)]
            + [pltpu.VMEM((rpt, LANES), jnp.float32)]
            + [pltpu.VMEM_SHARED((n_pad, LANES), jnp.float32)]
            + [pltpu.SemaphoreType.DMA] * (2 * NBUF + 1)
        ),
    )
    def agg_kernel(edge_hbm, z_hbm, out_hbm, *refs):
        rowv, colv = refs[0], refs[1]
        bufs = refs[2:2 + NBUF]
        tbuf = refs[2 + NBUF]
        zero_v = refs[3 + NBUF]
        acc_sh = refs[4 + NBUF]
        gsem = refs[5 + NBUF:5 + 2 * NBUF]
        ssem = refs[5 + 2 * NBUF:5 + 3 * NBUF]
        tsem = refs[5 + 3 * NBUF]

        cid = lax.axis_index("c")
        sid = lax.axis_index("s")
        wid = sid * NC + cid

        def fill_zero(i, carry):
            zero_v[i, :] = jnp.zeros((LANES,), jnp.float32)
            return carry
        lax.fori_loop(0, rpt, fill_zero, 0)

        nzt = nz // rpt
        zrem = nz - nzt * rpt

        @pl.when(cid == 0)
        def _():
            @pl.when(sid < nzt)
            def _():
                pltpu.sync_copy(z_hbm.at[pl.ds(sid * rpt, rpt)],
                                acc_sh.at[pl.ds(sid * rpt, rpt)])

            @pl.when(sid == nzt)
            def _():
                if zrem:
                    pltpu.sync_copy(
                        z_hbm.at[pl.ds(nzt * rpt, zrem)],
                        acc_sh.at[pl.ds(nzt * rpt, zrem)])
                pltpu.sync_copy(
                    zero_v.at[pl.ds(0, rpt - zrem)],
                    acc_sh.at[pl.ds(nzt * rpt + zrem, rpt - zrem)])

            @pl.when(sid > nzt)
            def _():
                pltpu.sync_copy(zero_v, acc_sh.at[pl.ds(sid * rpt, rpt)])

        @pl.when(cid != 0)
        def _():
            pltpu.sync_copy(zero_v, acc_sh.at[pl.ds(sid * rpt, rpt)])

        plsc.subcore_barrier()

        pltpu.sync_copy(edge_hbm.at[0, pl.ds(wid * epw, epw)], rowv)
        pltpu.sync_copy(edge_hbm.at[1, pl.ds(wid * epw, epw)], colv)

        def start_g(j, b):
            pltpu.async_copy(z_hbm.at[rowv.at[pl.ds(j * CHUNK, CHUNK)]],
                             bufs[b], gsem[b])

        def wait_g(j, b):
            pltpu.make_async_copy(z_hbm.at[rowv.at[pl.ds(j * CHUNK, CHUNK)]],
                                  bufs[b], gsem[b]).wait()

        def start_s(j, b):
            pltpu.async_copy(bufs[b],
                             acc_sh.at[colv.at[pl.ds(j * CHUNK, CHUNK)]],
                             ssem[b], add=True)

        def wait_s(j, b):
            pltpu.make_async_copy(bufs[b],
                                  acc_sh.at[colv.at[pl.ds(j * CHUNK, CHUNK)]],
                                  ssem[b]).wait()

        if tail:
            toff = nfull * CHUNK
            pltpu.async_copy(z_hbm.at[rowv.at[pl.ds(toff, tail)]],
                             tbuf, tsem).wait()
            pltpu.sync_copy(tbuf, acc_sh.at[colv.at[pl.ds(toff, tail)]],
                            add=True)

        for b in range(NBUF):
            start_g(b, b)

        def body(ii, carry):
            j0 = ii * NBUF
            for b in range(NBUF):
                wait_g(j0 + b, b)
                start_s(j0 + b, b)
            for b in range(NBUF):
                wait_s(j0 + b, b)
                start_g(j0 + NBUF + b, b)
            return carry
        lax.fori_loop(0, nfull // NBUF - 1, body, 0)

        j0 = nfull - NBUF
        for b in range(NBUF):
            wait_g(j0 + b, b)
            start_s(j0 + b, b)
        for b in range(NBUF):
            wait_s(j0 + b, b)

        plsc.subcore_barrier()
        pltpu.sync_copy(acc_sh.at[pl.ds(sid * rpt, rpt)],
                        out_hbm.at[pl.ds(cid * n_pad + sid * rpt, rpt)])

    return agg_kernel


def kernel(x, edge_index, W, b):
    n, f_in = x.shape
    f_out = W.shape[0]
    e = edge_index.shape[1]

    n_pad = ((n + 1 + NS * LANES - 1) // (NS * LANES)) * (NS * LANES)
    assert e % (NW * 8) == 0
    epw = e // NW

    def tc0_body(x_ref, w_ref, y_ref):
        y_ref[...] = lax.dot_general(x_ref[...], w_ref[...],
                                     (((1,), (1,)), ((), ())),
                                     preferred_element_type=jnp.float32)

    y = pl.pallas_call(
        tc0_body,
        out_shape=jax.ShapeDtypeStruct((n, f_out), jnp.float32),
    )(x, W)

    degp = _make_deg_kernel(n_pad, epw)(edge_index)

    def tc1_body(y_ref, degp_ref, z_ref):
        cnt = sum(degp_ref[pl.ds(c * n_pad, n)] for c in range(NC))
        dis = lax.rsqrt(cnt + 1.0)
        z_ref[...] = y_ref[...] * lax.broadcast_in_dim(dis, (n, f_out), (0,))

    z = pl.pallas_call(
        tc1_body,
        out_shape=jax.ShapeDtypeStruct((n, f_out), jnp.float32),
    )(y, degp)

    aggp = _make_agg_kernel(n_pad, epw, n)(edge_index, z)

    def tc2_body(aggp_ref, degp_ref, b_ref, o_ref):
        aggz = sum(aggp_ref[c * n_pad:c * n_pad + n, :] for c in range(NC))
        cnt = sum(degp_ref[pl.ds(c * n_pad, n)] for c in range(NC))
        dis = lax.broadcast_in_dim(lax.rsqrt(cnt + 1.0), (n, f_out), (0,))
        h = dis * aggz + b_ref[...]
        m = jnp.max(h, axis=1, keepdims=True)
        sh = h - m
        lse = jnp.log(jnp.sum(jnp.exp(sh), axis=1, keepdims=True))
        o_ref[...] = sh - lse

    return pl.pallas_call(
        tc2_body,
        out_shape=jax.ShapeDtypeStruct((n, f_out), jnp.float32),
    )(aggp, degp, b.reshape(1, f_out))
